# Initial kernel scaffold; baseline (speedup 1.0000x reference)
import jax
import jax.numpy as jnp
from jax.experimental import pallas as pl


def kernel(output_a):
    raise NotImplementedError("write your pallas kernel here")


# speedup vs baseline: 21.4351x; 21.4351x over previous
import jax
import jax.numpy as jnp
from jax.experimental import pallas as pl

_K = 1024
_ROWS_PER_BLOCK = 16


def _mask_kernel(x_ref, o_ref):
    sign = jnp.int32(-2**31)
    x = x_ref[...]
    bits = jax.lax.bitcast_convert_type(x, jnp.int32)
    xk = jnp.where(bits < 0, bits ^ jnp.int32(0x7FFFFFFF), bits)

    def body(i, pu):
        bit = jax.lax.shift_left(jnp.int32(1), jnp.int32(31) - i)
        t = pu | bit
        tx = t ^ sign
        cnt = jnp.sum((xk >= tx).astype(jnp.int32), axis=1, keepdims=True)
        return jnp.where(cnt >= _K, t, pu)

    pu0 = jnp.zeros((x.shape[0], 1), jnp.int32)
    pu = jax.lax.fori_loop(0, 32, body, pu0)
    tx = pu ^ sign
    o_ref[...] = (xk >= tx).astype(jnp.float32)


@jax.jit
def kernel(output_a):
    n_rows, n_cols = output_a.shape
    grid = (n_rows // _ROWS_PER_BLOCK,)
    return pl.pallas_call(
        _mask_kernel,
        grid=grid,
        in_specs=[pl.BlockSpec((_ROWS_PER_BLOCK, n_cols), lambda i: (i, 0))],
        out_specs=pl.BlockSpec((_ROWS_PER_BLOCK, n_cols), lambda i: (i, 0)),
        out_shape=jax.ShapeDtypeStruct((n_rows, n_cols), jnp.float32),
    )(output_a)


# speedup vs baseline: 28.4229x; 1.3260x over previous
import jax
import jax.numpy as jnp
from jax.experimental import pallas as pl

_K = 1024
_ROWS_PER_BLOCK = 64


def _mask_kernel(x_ref, o_ref):
    sign = jnp.int32(-2**31)
    x = x_ref[...]
    bits = jax.lax.bitcast_convert_type(x, jnp.int32)
    xk = jnp.where(bits < 0, bits ^ jnp.int32(0x7FFFFFFF), bits)

    def body(i, pu):
        bit = jax.lax.shift_left(jnp.int32(1), jnp.int32(31) - i)
        t = pu | bit
        tx = t ^ sign
        cnt = jnp.sum((xk >= tx).astype(jnp.int32), axis=1, keepdims=True)
        return jnp.where(cnt >= _K, t, pu)

    pu0 = jnp.zeros((x.shape[0], 1), jnp.int32)
    pu = jax.lax.fori_loop(0, 32, body, pu0)
    tx = pu ^ sign
    o_ref[...] = (xk >= tx).astype(jnp.float32)


@jax.jit
def kernel(output_a):
    n_rows, n_cols = output_a.shape
    grid = (n_rows // _ROWS_PER_BLOCK,)
    return pl.pallas_call(
        _mask_kernel,
        grid=grid,
        in_specs=[pl.BlockSpec((_ROWS_PER_BLOCK, n_cols), lambda i: (i, 0))],
        out_specs=pl.BlockSpec((_ROWS_PER_BLOCK, n_cols), lambda i: (i, 0)),
        out_shape=jax.ShapeDtypeStruct((n_rows, n_cols), jnp.float32),
    )(output_a)


# speedup vs baseline: 28.7111x; 1.0101x over previous
import jax
import jax.numpy as jnp
from jax.experimental import pallas as pl

_K = 1024
_ROWS_PER_BLOCK = 128


def _mask_kernel(x_ref, o_ref):
    sign = jnp.int32(-2**31)
    x = x_ref[...]
    bits = jax.lax.bitcast_convert_type(x, jnp.int32)
    xk = jnp.where(bits < 0, bits ^ jnp.int32(0x7FFFFFFF), bits)

    def body(i, pu):
        bit = jax.lax.shift_left(jnp.int32(1), jnp.int32(31) - i)
        t = pu | bit
        tx = t ^ sign
        cnt = jnp.sum((xk >= tx).astype(jnp.int32), axis=1, keepdims=True)
        return jnp.where(cnt >= _K, t, pu)

    pu0 = jnp.zeros((x.shape[0], 1), jnp.int32)
    pu = jax.lax.fori_loop(0, 32, body, pu0)
    tx = pu ^ sign
    o_ref[...] = (xk >= tx).astype(jnp.float32)


@jax.jit
def kernel(output_a):
    n_rows, n_cols = output_a.shape
    grid = (n_rows // _ROWS_PER_BLOCK,)
    return pl.pallas_call(
        _mask_kernel,
        grid=grid,
        in_specs=[pl.BlockSpec((_ROWS_PER_BLOCK, n_cols), lambda i: (i, 0))],
        out_specs=pl.BlockSpec((_ROWS_PER_BLOCK, n_cols), lambda i: (i, 0)),
        out_shape=jax.ShapeDtypeStruct((n_rows, n_cols), jnp.float32),
    )(output_a)
